# baseline (speedup 1.0000x reference)
import functools
import numpy as np
import jax
import jax.numpy as jnp
from jax import lax
from jax.experimental import pallas as pl
from jax.experimental.pallas import tpu as pltpu
from jax.experimental.pallas import tpu_sc as plsc

_G, _C, _GD, _R = 2, 4, 512, 2
_L = 16
_KC = 32
_KU = 8
_JH = 4

_V0 = np.float32(np.arctanh(np.float64(0.5) / 2.002))
_V1 = np.float32(np.arctanh(np.float64(1.5) / 2.002))


def _steps(v):
    one = jnp.float32(1.0)
    zero = jnp.float32(0.0)
    s = jnp.where(v > _V0, one, zero)
    s = s + jnp.where(v >= _V1, one, zero)
    s = s + jnp.where(v >= -_V0, one, zero)
    s = s + jnp.where(v > -_V1, one, zero)
    return s


def _rne_bf16(x):
    u = lax.bitcast_convert_type(x, jnp.uint32)
    u = (u + jnp.uint32(0x7FFF) + ((u >> 16) & jnp.uint32(1))) & jnp.uint32(
        0xFFFF0000)
    return lax.bitcast_convert_type(u, jnp.float32)


def _rhu_bf16(x):
    u = lax.bitcast_convert_type(x, jnp.uint32)
    u = (u + jnp.uint32(0x8000)) & jnp.uint32(0xFFFF0000)
    return lax.bitcast_convert_type(u, jnp.float32)


def _make_sc_body(BG, T):
    WPB = 32 // BG
    TH = T // WPB
    NLG = TH // (_JH * _L)
    NCH = _GD // _KC

    def body(x_hbm, w_hbm, bias_hbm, out_hbm,
             wbuf, bbuf, xbuf, zbuf, obuf, xsem0, xsem1):
        cid = lax.axis_index("c")
        sid = lax.axis_index("s")
        wid = cid * 16 + sid
        bg = wid // WPB
        th = wid % WPB
        g = bg % _G
        tbase = th * TH

        pltpu.sync_copy(w_hbm.at[g], wbuf)
        pltpu.sync_copy(bias_hbm.at[g], bbuf)

        def xcopy(ci, slot, sem):
            return pltpu.make_async_copy(
                x_hbm.at[bg, pl.ds(ci * _KC, _KC), pl.ds(tbase, TH)],
                xbuf.at[slot], sem)

        xcopy(0, 0, xsem0).start()
        xcopy(1, 1, xsem1).start()

        def init_body(i, carry):
            for c in range(_C):
                zbuf[c, pl.ds(i * _L, _L)] = bbuf[c]
            return carry

        lax.fori_loop(0, TH // _L, init_body, 0)

        def pair_body(jp, carry):
            for slot in range(2):
                ci = jp * 2 + slot
                xsem = xsem0 if slot == 0 else xsem1
                xcopy(ci, slot, xsem).wait()
                k0 = ci * _KC

                def lgb_body(lgb, carry2):
                    t0 = lgb * (_JH * _L)
                    accs = [zbuf[c, pl.ds(t0 + j * _L, _L)]
                            for c in range(_C) for j in range(_JH)]

                    def kbody(kb, accs):
                        accs = list(accs)
                        for kk in range(_KU):
                            k = kb * _KU + kk
                            wvs = [wbuf[c, pl.ds((k0 + k) * _L, _L)]
                                   for c in range(_C)]
                            for j in range(_JH):
                                xv = _rhu_bf16(
                                    xbuf[slot, k, pl.ds(t0 + j * _L, _L)])
                                for c in range(_C):
                                    accs[c * _JH + j] = (
                                        accs[c * _JH + j] + xv * wvs[c])
                        return tuple(accs)

                    accs = lax.fori_loop(0, _KC // _KU, kbody, tuple(accs))
                    for c in range(_C):
                        for j in range(_JH):
                            zbuf[c, pl.ds(t0 + j * _L, _L)] = (
                                accs[c * _JH + j])
                    return carry2

                lax.fori_loop(0, NLG, lgb_body, 0)

                @pl.when(ci + 2 < NCH)
                def _():
                    xcopy(ci + 2, slot, xsem).start()
            return carry

        lax.fori_loop(0, NCH // 2, pair_body, 0)

        def fsq_body(lgb, carry):
            t0 = lgb * (_JH * _L)
            for j in range(_JH):
                idx0 = None
                idx1 = None
                for c in range(_C):
                    z = zbuf[c, pl.ds(t0 + j * _L, _L)]
                    s0 = _steps(z)
                    a1 = 4.0 * z - 2.0 * (s0 - 2.0)
                    s1 = _steps(a1)
                    w5 = jnp.float32(5.0 ** c)
                    c0 = s0 * w5
                    c1 = s1 * w5
                    idx0 = c0 if idx0 is None else idx0 + c0
                    idx1 = c1 if idx1 is None else idx1 + c1
                obuf[0, pl.ds(t0 + j * _L, _L)] = idx0.astype(jnp.int32)
                obuf[1, pl.ds(t0 + j * _L, _L)] = idx1.astype(jnp.int32)
            return carry

        lax.fori_loop(0, NLG, fsq_body, 0)
        pltpu.sync_copy(obuf, out_hbm.at[bg, :, pl.ds(tbase, TH)])

    return body


def _sc_kernel(x, Win, b_in):
    B, DIM, T = x.shape
    x2 = x.reshape(B * _G, _GD, T)
    Wr = _rne_bf16(Win)
    Wbc = jnp.broadcast_to(
        Wr[:, :, :, None], (_G, _C, _GD, _L)).reshape(_G, _C, _GD * _L)
    bias_bc = jnp.broadcast_to(b_in[:, :, None], (_G, _C, _L))
    BG = B * _G
    TH = T // (32 // BG)
    mesh = plsc.VectorSubcoreMesh(core_axis_name="c", subcore_axis_name="s")
    run = functools.partial(
        pl.kernel,
        mesh=mesh,
        out_type=jax.ShapeDtypeStruct((BG, _R, T), jnp.int32),
        scratch_types=[
            pltpu.VMEM((_C, _GD * _L), jnp.float32),
            pltpu.VMEM((_C, _L), jnp.float32),
            pltpu.VMEM((2, _KC, TH), jnp.float32),
            pltpu.VMEM((_C, TH), jnp.float32),
            pltpu.VMEM((_R, TH), jnp.int32),
            pltpu.SemaphoreType.DMA,
            pltpu.SemaphoreType.DMA,
        ],
    )(_make_sc_body(BG, T))
    out = run(x2, Wbc, bias_bc)
    return out.reshape(B, _G * _R, T)


@jax.jit
def kernel(x, Win, b_in):
    return _sc_kernel(x, Win, b_in)


# speedup vs baseline: 1.7663x; 1.7663x over previous
import functools
import numpy as np
import jax
import jax.numpy as jnp
from jax import lax
from jax.experimental import pallas as pl
from jax.experimental.pallas import tpu as pltpu
from jax.experimental.pallas import tpu_sc as plsc

_G, _C, _GD, _R = 2, 4, 512, 2
_L = 16
_KC = 32
_KU = 4
_JH = 4

_V0 = np.float32(np.arctanh(np.float64(0.5) / 2.002))
_V1 = np.float32(np.arctanh(np.float64(1.5) / 2.002))


def _steps(v):
    one = jnp.float32(1.0)
    zero = jnp.float32(0.0)
    s = jnp.where(v > _V0, one, zero)
    s = s + jnp.where(v >= _V1, one, zero)
    s = s + jnp.where(v >= -_V0, one, zero)
    s = s + jnp.where(v > -_V1, one, zero)
    return s


def _rne_bf16(x):
    u = lax.bitcast_convert_type(x, jnp.uint32)
    u = (u + jnp.uint32(0x7FFF) + ((u >> 16) & jnp.uint32(1))) & jnp.uint32(
        0xFFFF0000)
    return lax.bitcast_convert_type(u, jnp.float32)


def _rhu_bf16(x):
    u = lax.bitcast_convert_type(x, jnp.uint32)
    u = (u + jnp.uint32(0x8000)) & jnp.uint32(0xFFFF0000)
    return lax.bitcast_convert_type(u, jnp.float32)


def _make_sc_body(BG, T):
    WPB = 32 // BG
    TH = T // WPB
    NLG = TH // (_JH * _L)
    NCH = _GD // _KC

    def body(x_hbm, w_hbm, bias_hbm, out_hbm,
             wbuf, bbuf, xbuf, zbuf, obuf, xsem0, xsem1):
        cid = lax.axis_index("c")
        sid = lax.axis_index("s")
        wid = cid * 16 + sid
        bg = wid // WPB
        th = wid % WPB
        g = bg % _G
        tbase = th * TH

        pltpu.sync_copy(w_hbm.at[g], wbuf)
        pltpu.sync_copy(bias_hbm.at[g], bbuf)

        def xcopy(ci, slot, sem):
            return pltpu.make_async_copy(
                x_hbm.at[bg, pl.ds(ci * _KC, _KC), pl.ds(tbase, TH)],
                xbuf.at[slot], sem)

        xcopy(0, 0, xsem0).start()
        xcopy(1, 1, xsem1).start()

        def init_body(i, carry):
            for c in range(_C):
                zbuf[c, pl.ds(i * _L, _L)] = bbuf[c]
            return carry

        lax.fori_loop(0, TH // _L, init_body, 0)

        def pair_body(jp, carry):
            for slot in range(2):
                ci = jp * 2 + slot
                xsem = xsem0 if slot == 0 else xsem1
                xcopy(ci, slot, xsem).wait()
                k0 = ci * _KC

                def lgb_body(lgb, carry2):
                    t0 = lgb * (_JH * _L)
                    accs = [zbuf[c, pl.ds(t0 + j * _L, _L)]
                            for c in range(_C) for j in range(_JH)]

                    def kbody(kb, accs):
                        accs = list(accs)
                        for kk in range(_KU):
                            k = kb * _KU + kk
                            wvs = [wbuf[c, pl.ds((k0 + k) * _L, _L)]
                                   for c in range(_C)]
                            for j in range(_JH):
                                xv = _rhu_bf16(
                                    xbuf[slot, k, pl.ds(t0 + j * _L, _L)])
                                for c in range(_C):
                                    accs[c * _JH + j] = (
                                        accs[c * _JH + j] + xv * wvs[c])
                        return tuple(accs)

                    accs = lax.fori_loop(0, _KC // _KU, kbody, tuple(accs))
                    for c in range(_C):
                        for j in range(_JH):
                            zbuf[c, pl.ds(t0 + j * _L, _L)] = (
                                accs[c * _JH + j])
                    return carry2

                lax.fori_loop(0, NLG, lgb_body, 0)

                @pl.when(ci + 2 < NCH)
                def _():
                    xcopy(ci + 2, slot, xsem).start()
            return carry

        lax.fori_loop(0, NCH // 2, pair_body, 0)

        def fsq_body(lgb, carry):
            t0 = lgb * (_JH * _L)
            for j in range(_JH):
                idx0 = None
                idx1 = None
                for c in range(_C):
                    z = zbuf[c, pl.ds(t0 + j * _L, _L)]
                    s0 = _steps(z)
                    a1 = 4.0 * z - 2.0 * (s0 - 2.0)
                    s1 = _steps(a1)
                    w5 = jnp.float32(5.0 ** c)
                    c0 = s0 * w5
                    c1 = s1 * w5
                    idx0 = c0 if idx0 is None else idx0 + c0
                    idx1 = c1 if idx1 is None else idx1 + c1
                obuf[0, pl.ds(t0 + j * _L, _L)] = idx0.astype(jnp.int32)
                obuf[1, pl.ds(t0 + j * _L, _L)] = idx1.astype(jnp.int32)
            return carry

        lax.fori_loop(0, NLG, fsq_body, 0)
        pltpu.sync_copy(obuf, out_hbm.at[bg, :, pl.ds(tbase, TH)])

    return body


def _sc_kernel(x, Win, b_in):
    B, DIM, T = x.shape
    x2 = x.reshape(B * _G, _GD, T)
    Wr = _rne_bf16(Win)
    Wbc = jnp.broadcast_to(
        Wr[:, :, :, None], (_G, _C, _GD, _L)).reshape(_G, _C, _GD * _L)
    bias_bc = jnp.broadcast_to(b_in[:, :, None], (_G, _C, _L))
    BG = B * _G
    TH = T // (32 // BG)
    mesh = plsc.VectorSubcoreMesh(core_axis_name="c", subcore_axis_name="s")
    run = functools.partial(
        pl.kernel,
        mesh=mesh,
        out_type=jax.ShapeDtypeStruct((BG, _R, T), jnp.int32),
        scratch_types=[
            pltpu.VMEM((_C, _GD * _L), jnp.float32),
            pltpu.VMEM((_C, _L), jnp.float32),
            pltpu.VMEM((2, _KC, TH), jnp.float32),
            pltpu.VMEM((_C, TH), jnp.float32),
            pltpu.VMEM((_R, TH), jnp.int32),
            pltpu.SemaphoreType.DMA,
            pltpu.SemaphoreType.DMA,
        ],
    )(_make_sc_body(BG, T))
    out = run(x2, Wbc, bias_bc)
    return out.reshape(B, _G * _R, T)


@jax.jit
def kernel(x, Win, b_in):
    return _sc_kernel(x, Win, b_in)
